# baseline (speedup 1.0000x reference)
import functools

import jax
import jax.numpy as jnp
from jax import lax
from jax.experimental import pallas as pl
from jax.experimental.pallas import tpu as pltpu
from jax.experimental.pallas import tpu_sc as plsc

HIDDEN = 768
LANES = 16
NCH = HIDDEN // LANES
EPS = 1e-5
N_TOKENS = 4 * 2048
NUM_WORKERS = 32
CHUNK = 32
PIECES = 2
TOK_PER_PIECE = N_TOKENS // PIECES
ROW_BLOCK = 1024
BLOCKS_PER_PIECE = TOK_PER_PIECE // ROW_BLOCK


def _make_gather_sum(tokens):
    tok_per_worker = tokens // NUM_WORKERS
    nchunks = tok_per_worker // CHUNK

    def body(ids_hbm, pids_hbm, word_hbm, pos_hbm, x_hbm,
             idw0, idw1, idp0, idp1, w0, w1, p0, p1,
             sw0, sw1, sp0, sp1, so0, so1):
        wid = lax.axis_index("s") * 2 + lax.axis_index("c")
        base = wid * tok_per_worker

        idw = (idw0, idw1)
        idp = (idp0, idp1)
        wr = (w0, w1)
        pr = (p0, p1)
        sw = (sw0, sw1)
        sp = (sp0, sp1)
        so = (so0, so1)

        gather_h = [None, None]
        out_h = [None, None]

        def start_gather(c):
            b = c % 2
            off = base + c * CHUNK
            pltpu.sync_copy(ids_hbm.at[pl.ds(off, CHUNK)], idw[b])
            pltpu.sync_copy(pids_hbm.at[pl.ds(off, CHUNK)], idp[b])
            hw = pltpu.async_copy(word_hbm.at[idw[b]], wr[b], sw[b])
            hp = pltpu.async_copy(pos_hbm.at[idp[b]], pr[b], sp[b])
            gather_h[b] = (hw, hp)

        start_gather(0)
        for c in range(nchunks):
            b = c % 2
            hw, hp = gather_h[b]
            hw.wait()
            hp.wait()
            if c + 1 < nchunks:
                if out_h[1 - b] is not None:
                    out_h[1 - b].wait()
                start_gather(c + 1)

            wb, pb = wr[b], pr[b]

            def sum_body(t, carry, wb=wb, pb=pb):
                for j in range(NCH):
                    sl = pl.ds(j * LANES, LANES)
                    wb[t, sl] = wb[t, sl] + pb[t, sl]
                return carry

            lax.fori_loop(0, CHUNK, sum_body, 0)
            off = base + c * CHUNK
            out_h[b] = pltpu.async_copy(wb, x_hbm.at[pl.ds(off, CHUNK)], so[b])
        for b in (0, 1):
            if out_h[b] is not None:
                out_h[b].wait()

    mesh = plsc.VectorSubcoreMesh(core_axis_name="c", subcore_axis_name="s")
    return functools.partial(
        pl.kernel,
        mesh=mesh,
        compiler_params=pltpu.CompilerParams(needs_layout_passes=False),
        out_type=jax.ShapeDtypeStruct((tokens, HIDDEN), jnp.float32),
        scratch_types=[
            pltpu.VMEM((CHUNK,), jnp.int32),
            pltpu.VMEM((CHUNK,), jnp.int32),
            pltpu.VMEM((CHUNK,), jnp.int32),
            pltpu.VMEM((CHUNK,), jnp.int32),
            pltpu.VMEM((CHUNK, HIDDEN), jnp.float32),
            pltpu.VMEM((CHUNK, HIDDEN), jnp.float32),
            pltpu.VMEM((CHUNK, HIDDEN), jnp.float32),
            pltpu.VMEM((CHUNK, HIDDEN), jnp.float32),
            pltpu.SemaphoreType.DMA,
            pltpu.SemaphoreType.DMA,
            pltpu.SemaphoreType.DMA,
            pltpu.SemaphoreType.DMA,
            pltpu.SemaphoreType.DMA,
            pltpu.SemaphoreType.DMA,
        ],
    )(body)


_gather_sum_piece = _make_gather_sum(TOK_PER_PIECE)


def _ln_math(tidf, tt, scale, bias, x):
    t0 = tt[0:1, :]
    d = tt[1:2, :] - t0
    x = x + t0 + tidf * d
    mean = jnp.mean(x, axis=1, keepdims=True)
    xc = x - mean
    var = jnp.mean(xc * xc, axis=1, keepdims=True)
    return xc * lax.rsqrt(var + EPS) * scale + bias


def _ln_body(tidf_ref, tt_ref, scale_ref, bias_ref, x_ref, o_ref):
    o_ref[...] = _ln_math(tidf_ref[...], tt_ref[...], scale_ref[...],
                          bias_ref[...], x_ref[...])


def _ln_body_prev(tidf_ref, tt_ref, scale_ref, bias_ref, x_ref, prev_ref,
                  o_ref):
    del prev_ref
    o_ref[...] = _ln_math(tidf_ref[...], tt_ref[...], scale_ref[...],
                          bias_ref[...], x_ref[...])


def _ln_piece(x_piece, tidf_piece, tt, scale2d, bias2d, prev, piece):
    in_specs = [
        pl.BlockSpec((ROW_BLOCK, 1), lambda i: (i, 0)),
        pl.BlockSpec((2, HIDDEN), lambda i: (0, 0)),
        pl.BlockSpec((1, HIDDEN), lambda i: (0, 0)),
        pl.BlockSpec((1, HIDDEN), lambda i: (0, 0)),
        pl.BlockSpec((ROW_BLOCK, HIDDEN), lambda i: (i, 0)),
    ]
    ins = [tidf_piece, tt, scale2d, bias2d, x_piece]
    if prev is None:
        body = _ln_body
        aliases = {}
    else:
        body = _ln_body_prev
        in_specs.append(pl.BlockSpec(memory_space=pl.ANY))
        ins.append(prev)
        aliases = {5: 0}
    return pl.pallas_call(
        body,
        grid=(BLOCKS_PER_PIECE,),
        in_specs=in_specs,
        out_specs=pl.BlockSpec((ROW_BLOCK, HIDDEN),
                               lambda i, p=piece: (i + p * BLOCKS_PER_PIECE, 0)),
        out_shape=jax.ShapeDtypeStruct((N_TOKENS, HIDDEN), jnp.float32),
        input_output_aliases=aliases,
    )(*ins)


def kernel(input_ids, token_type_ids, position_ids, attention_mask,
           word_embeddings, position_embeddings, token_type_embeddings,
           ln_scale, ln_bias):
    del attention_mask
    ids = input_ids.reshape(-1).astype(jnp.int32)
    pids = position_ids.reshape(-1).astype(jnp.int32)
    tidf = token_type_ids.reshape(-1, 1).astype(jnp.float32)
    scale2d = ln_scale.reshape(1, HIDDEN)
    bias2d = ln_bias.reshape(1, HIDDEN)

    out = None
    for p in range(PIECES):
        sl = slice(p * TOK_PER_PIECE, (p + 1) * TOK_PER_PIECE)
        x_p = _gather_sum_piece(ids[sl], pids[sl], word_embeddings,
                                position_embeddings)
        out = _ln_piece(x_p, tidf[sl], token_type_embeddings,
                        scale2d, bias2d, out, p)
    return out.reshape(input_ids.shape + (HIDDEN,))
